# Initial kernel scaffold; baseline (speedup 1.0000x reference)
import jax
import jax.numpy as jnp
from jax.experimental import pallas as pl


def kernel(x, adj, H_, params):
    raise NotImplementedError("write your pallas kernel here")


# speedup vs baseline: 1.1470x; 1.1470x over previous
import functools
import jax
import jax.numpy as jnp
from jax.experimental import pallas as pl
from jax.experimental.pallas import tpu as pltpu

MODES = 2
NODES = 2500
B = 4
SL = 128
NT = NODES * B
HD = 6
GH = 16
EM = 4


def _mm_kernel(x_ref, w_ref, b_ref, o_ref, *, act, nk):
    k = pl.program_id(1)

    @pl.when(k == 0)
    def _():
        o_ref[...] = jnp.zeros_like(o_ref)

    o_ref[...] += jnp.dot(x_ref[...], w_ref[...],
                          preferred_element_type=jnp.float32)

    @pl.when(k == nk - 1)
    def _():
        acc = o_ref[...] + b_ref[...]
        if act == 'relu':
            acc = jnp.maximum(acc, 0.0)
        elif act == 'sigmoid':
            acc = jax.nn.sigmoid(acc)
        o_ref[...] = acc


def matmul(x, w, b=None, act='none', bm=None, bk=None):
    m, kdim = x.shape
    n = w.shape[1]
    if b is None:
        b = jnp.zeros((n,), jnp.float32)
    b = b.reshape(1, n)
    bm = bm or m
    bk = bk or kdim
    grid = (m // bm, kdim // bk)
    return pl.pallas_call(
        functools.partial(_mm_kernel, act=act, nk=grid[1]),
        grid=grid,
        in_specs=[
            pl.BlockSpec((bm, bk), lambda i, k: (i, k)),
            pl.BlockSpec((bk, n), lambda i, k: (k, 0)),
            pl.BlockSpec((1, n), lambda i, k: (0, 0)),
        ],
        out_specs=pl.BlockSpec((bm, n), lambda i, k: (i, 0)),
        out_shape=jax.ShapeDtypeStruct((m, n), jnp.float32),
    )(x, w, b)


def _mlp3_kernel(x_ref, w1, b1, w2, b2, w3, b3, o_ref, *, final_act):
    h = jnp.maximum(jnp.dot(x_ref[...], w1[...],
                            preferred_element_type=jnp.float32) + b1[...], 0.0)
    h = jnp.maximum(jnp.dot(h, w2[...],
                            preferred_element_type=jnp.float32) + b2[...], 0.0)
    h = jnp.dot(h, w3[...], preferred_element_type=jnp.float32) + b3[...]
    if final_act == 'sigmoid':
        h = jax.nn.sigmoid(h)
    o_ref[...] = h


def mlp3(x, w1, b1, w2, b2, w3, b3, final_act='none', bm=2000):
    m, kdim = x.shape
    d1 = w1.shape[1]
    d2 = w2.shape[1]
    n = w3.shape[1]
    grid = (m // bm,)
    full = lambda a: pl.BlockSpec(a.shape, lambda i: tuple(0 for _ in a.shape))
    b1r, b2r, b3r = b1.reshape(1, d1), b2.reshape(1, d2), b3.reshape(1, n)
    return pl.pallas_call(
        functools.partial(_mlp3_kernel, final_act=final_act),
        grid=grid,
        in_specs=[
            pl.BlockSpec((bm, kdim), lambda i: (i, 0)),
            full(w1), full(b1r), full(w2), full(b2r), full(w3), full(b3r),
        ],
        out_specs=pl.BlockSpec((bm, n), lambda i: (i, 0)),
        out_shape=jax.ShapeDtypeStruct((m, n), jnp.float32),
    )(x, w1, b1r, w2, b2r, w3, b3r)


def _cf_kernel(x_ref, w1_ref, b1, w2, b2, w3, b3, o_ref, acc, *, nk):
    k = pl.program_id(0)

    @pl.when(k == 0)
    def _():
        acc[...] = jnp.zeros_like(acc)

    acc[...] += jnp.dot(x_ref[...], w1_ref[...],
                        preferred_element_type=jnp.float32)

    @pl.when(k == nk - 1)
    def _():
        h = jnp.maximum(acc[...] + b1[...], 0.0)
        h = jnp.maximum(jnp.dot(h, w2[...],
                                preferred_element_type=jnp.float32) + b2[...],
                        0.0)
        h = jnp.dot(h, w3[...], preferred_element_type=jnp.float32) + b3[...]
        o_ref[...] = jax.nn.sigmoid(h)


def classifier(flat, w1, b1, w2, b2, w3, b3, bk=4096):
    kp = ((flat.shape[1] + bk - 1) // bk) * bk
    flat = jnp.pad(flat, ((0, 0), (0, kp - flat.shape[1])))
    w1 = jnp.pad(w1, ((0, kp - w1.shape[0]), (0, 0)))
    m, kdim = flat.shape
    d1 = w1.shape[1]
    nk = kdim // bk
    full = lambda a: pl.BlockSpec(a.shape, lambda k: tuple(0 for _ in a.shape))
    b1r, b2r, b3r = b1.reshape(1, -1), b2.reshape(1, -1), b3.reshape(1, -1)
    return pl.pallas_call(
        functools.partial(_cf_kernel, nk=nk),
        grid=(nk,),
        in_specs=[
            pl.BlockSpec((m, bk), lambda k: (0, k)),
            pl.BlockSpec((bk, d1), lambda k: (k, 0)),
            full(b1r), full(w2), full(b2r), full(w3), full(b3r),
        ],
        out_specs=pl.BlockSpec((m, 2), lambda k: (0, 0)),
        out_shape=jax.ShapeDtypeStruct((m, 2), jnp.float32),
        scratch_shapes=[pltpu.VMEM((m, d1), jnp.float32)],
    )(flat, w1, b1r, w2, b2r, w3, b3r)


def _gru_kernel(gi0_ref, h0_ref, h1_ref, whh0, bhh0, wih1, bih1, whh1, bhh1,
                y_ref, h0o_ref, h1o_ref, *, T):
    def step(t, carry):
        h0, h1 = carry
        gi = gi0_ref[pl.ds(t, 1)][0]
        gh = jnp.dot(h0, whh0[...], preferred_element_type=jnp.float32) \
            + bhh0[...]
        i_r, i_z, i_n = jnp.split(gi, 3, axis=-1)
        h_r, h_z, h_n = jnp.split(gh, 3, axis=-1)
        r = jax.nn.sigmoid(i_r + h_r)
        z = jax.nn.sigmoid(i_z + h_z)
        nn = jnp.tanh(i_n + r * h_n)
        h0n = (1.0 - z) * nn + z * h0

        gi1 = jnp.dot(h0n, wih1[...], preferred_element_type=jnp.float32) \
            + bih1[...]
        gh1 = jnp.dot(h1, whh1[...], preferred_element_type=jnp.float32) \
            + bhh1[...]
        i_r1, i_z1, i_n1 = jnp.split(gi1, 3, axis=-1)
        h_r1, h_z1, h_n1 = jnp.split(gh1, 3, axis=-1)
        r1 = jax.nn.sigmoid(i_r1 + h_r1)
        z1 = jax.nn.sigmoid(i_z1 + h_z1)
        nn1 = jnp.tanh(i_n1 + r1 * h_n1)
        h1n = (1.0 - z1) * nn1 + z1 * h1

        y_ref[pl.ds(t, 1)] = h1n[None]
        return (h0n, h1n)

    h0, h1 = jax.lax.fori_loop(0, T, step, (h0_ref[...], h1_ref[...]))
    h0o_ref[...] = h0
    h1o_ref[...] = h1


def gru2(gi0, h00, h01, whh0, bhh0, wih1, bih1, whh1, bhh1):
    T = gi0.shape[0]
    full = lambda a: pl.BlockSpec(a.shape, lambda: tuple(0 for _ in a.shape))
    args = [gi0, h00, h01, whh0.T, bhh0.reshape(1, -1), wih1.T,
            bih1.reshape(1, -1), whh1.T, bhh1.reshape(1, -1)]
    return pl.pallas_call(
        functools.partial(_gru_kernel, T=T),
        in_specs=[full(a) for a in args],
        out_specs=[
            pl.BlockSpec((T, B, GH), lambda: (0, 0, 0)),
            pl.BlockSpec((B, GH), lambda: (0, 0)),
            pl.BlockSpec((B, GH), lambda: (0, 0)),
        ],
        out_shape=[
            jax.ShapeDtypeStruct((T, B, GH), jnp.float32),
            jax.ShapeDtypeStruct((B, GH), jnp.float32),
            jax.ShapeDtypeStruct((B, GH), jnp.float32),
        ],
    )(*args)


def _edge_aggregate(h, als, ald, s, d, heads, outc):
    e = als[s] + ald[d]
    e = jnp.where(e >= 0, e, 0.2 * e)
    w = jnp.exp(e)
    den = jax.ops.segment_sum(w, d, num_segments=NT)
    hh = h.reshape(NT, heads, outc)
    num = jax.ops.segment_sum(hh[s] * w[:, :, None], d, num_segments=NT)
    out = num / (den + 1e-16)[:, :, None]
    return out.reshape(NT, heads * outc)


def _gat(xh, s, d, W, a_s, a_d, bias, heads, outc):
    fin = W.shape[0]
    F = heads * outc
    As = jnp.zeros((F, HD + 2), jnp.float32)
    Ad = jnp.zeros((F, HD + 2), jnp.float32)
    As = As.at[jnp.arange(F), jnp.repeat(jnp.arange(heads), outc)].set(
        a_s.reshape(F))
    Ad = Ad.at[jnp.arange(F), jnp.repeat(jnp.arange(heads), outc)].set(
        a_d.reshape(F))
    Wcat = jnp.concatenate([W, W @ As, W @ Ad], axis=1)
    hcat = matmul(xh, Wcat, bm=2000)
    h = hcat[:, :F]
    als = hcat[:, F:F + heads]
    ald = hcat[:, F + HD + 2:F + HD + 2 + heads]
    out = _edge_aggregate(h, als, ald, s, d, heads, outc)
    return out + bias


def kernel(x, adj, H_, params):
    p = params
    loop = jnp.arange(NT, dtype=jnp.int32)
    sd = []
    for m in range(MODES):
        sd.append((jnp.concatenate([adj[m, 0], loop]),
                   jnp.concatenate([adj[m, 1], loop])))

    cat_t = None
    for m in range(MODES):
        mt = jax.lax.dynamic_slice_in_dim(x, m * NT, NT, 0)
        s, d = sd[m]
        g = _gat(mt, s, d, p['g1_W'][m], p['g1_as'][m], p['g1_ad'][m],
                 p['g1_b'][m], HD, 32)
        g = _gat(g, s, d, p['g2_W'][m], p['g2_as'][m], p['g2_ad'][m],
                 p['g2_b'][m], 1, EM)
        lf = mlp3(mt, p['nf_W1'], p['nf_b1'], p['nf_W2'], p['nf_b2'],
                  p['nf_W3'], p['nf_b3'])
        cat = matmul(jnp.concatenate([g, lf], axis=1), p['catP'][m])
        cat_t = cat if cat_t is None else cat_t + cat

    gi0 = matmul(cat_t, p['gru_Wih0'].T, p['gru_bih0'])
    gi0 = gi0.reshape(B, NODES, 3 * GH).transpose(1, 0, 2)
    y, h0T, h1T = gru2(gi0, H_[0], H_[1], p['gru_Whh0'], p['gru_bhh0'],
                       p['gru_Wih1'], p['gru_bih1'], p['gru_Whh1'],
                       p['gru_bhh1'])
    new_H = jnp.stack([h0T, h1T], axis=0)

    flat = y.transpose(1, 0, 2).reshape(B, NODES * GH)
    cf_out = classifier(flat, p['cf_W1'], p['cf_b1'], p['cf_W2'], p['cf_b2'],
                        p['cf_W3'], p['cf_b3'])

    rl = mlp3(cat_t, p['ml_W1'], p['ml_b1'], p['ml_W2'], p['ml_b2'],
              p['ml_W3'], p['ml_b3'])

    recs = []
    for m in range(MODES):
        s, d = sd[m]
        r = _gat(rl, s, d, p['r1_W'][m], p['r1_as'][m], p['r1_ad'][m],
                 p['r1_b'][m], HD, 32)
        r = _gat(r, s, d, p['r2_W'][m], p['r2_as'][m], p['r2_ad'][m],
                 p['r2_b'][m], 1, SL)
        recs.append(r)
    rec_out = jnp.concatenate(recs, axis=0)
    return (cf_out, rec_out, new_H)


# speedup vs baseline: 33.3638x; 29.0881x over previous
import functools
import jax
import jax.numpy as jnp
from jax.experimental import pallas as pl
from jax.experimental.pallas import tpu as pltpu
from jax.experimental.pallas import tpu_sc as plsc

MODES = 2
NODES = 2500
B = 4
SL = 128
NT = NODES * B
HD = 6
GH = 16
EM = 4


def _mm_kernel(x_ref, w_ref, b_ref, o_ref, *, act, nk):
    k = pl.program_id(1)

    @pl.when(k == 0)
    def _():
        o_ref[...] = jnp.zeros_like(o_ref)

    o_ref[...] += jnp.dot(x_ref[...], w_ref[...],
                          preferred_element_type=jnp.float32)

    @pl.when(k == nk - 1)
    def _():
        acc = o_ref[...] + b_ref[...]
        if act == 'relu':
            acc = jnp.maximum(acc, 0.0)
        elif act == 'sigmoid':
            acc = jax.nn.sigmoid(acc)
        o_ref[...] = acc


def matmul(x, w, b=None, act='none', bm=None, bk=None):
    m, kdim = x.shape
    n = w.shape[1]
    if b is None:
        b = jnp.zeros((n,), jnp.float32)
    b = b.reshape(1, n)
    bm = bm or m
    bk = bk or kdim
    grid = (m // bm, kdim // bk)
    return pl.pallas_call(
        functools.partial(_mm_kernel, act=act, nk=grid[1]),
        grid=grid,
        in_specs=[
            pl.BlockSpec((bm, bk), lambda i, k: (i, k)),
            pl.BlockSpec((bk, n), lambda i, k: (k, 0)),
            pl.BlockSpec((1, n), lambda i, k: (0, 0)),
        ],
        out_specs=pl.BlockSpec((bm, n), lambda i, k: (i, 0)),
        out_shape=jax.ShapeDtypeStruct((m, n), jnp.float32),
    )(x, w, b)


def _mlp3_kernel(x_ref, w1, b1, w2, b2, w3, b3, o_ref, *, final_act):
    h = jnp.maximum(jnp.dot(x_ref[...], w1[...],
                            preferred_element_type=jnp.float32) + b1[...], 0.0)
    h = jnp.maximum(jnp.dot(h, w2[...],
                            preferred_element_type=jnp.float32) + b2[...], 0.0)
    h = jnp.dot(h, w3[...], preferred_element_type=jnp.float32) + b3[...]
    if final_act == 'sigmoid':
        h = jax.nn.sigmoid(h)
    o_ref[...] = h


def mlp3(x, w1, b1, w2, b2, w3, b3, final_act='none', bm=2000):
    m, kdim = x.shape
    d1 = w1.shape[1]
    d2 = w2.shape[1]
    n = w3.shape[1]
    grid = (m // bm,)
    full = lambda a: pl.BlockSpec(a.shape, lambda i: tuple(0 for _ in a.shape))
    b1r, b2r, b3r = b1.reshape(1, d1), b2.reshape(1, d2), b3.reshape(1, n)
    return pl.pallas_call(
        functools.partial(_mlp3_kernel, final_act=final_act),
        grid=grid,
        in_specs=[
            pl.BlockSpec((bm, kdim), lambda i: (i, 0)),
            full(w1), full(b1r), full(w2), full(b2r), full(w3), full(b3r),
        ],
        out_specs=pl.BlockSpec((bm, n), lambda i: (i, 0)),
        out_shape=jax.ShapeDtypeStruct((m, n), jnp.float32),
    )(x, w1, b1r, w2, b2r, w3, b3r)


def _cf_kernel(x_ref, w1_ref, b1, w2, b2, w3, b3, o_ref, acc, *, nk):
    k = pl.program_id(0)

    @pl.when(k == 0)
    def _():
        acc[...] = jnp.zeros_like(acc)

    acc[...] += jnp.dot(x_ref[...], w1_ref[...],
                        preferred_element_type=jnp.float32)

    @pl.when(k == nk - 1)
    def _():
        h = jnp.maximum(acc[...] + b1[...], 0.0)
        h = jnp.maximum(jnp.dot(h, w2[...],
                                preferred_element_type=jnp.float32) + b2[...],
                        0.0)
        h = jnp.dot(h, w3[...], preferred_element_type=jnp.float32) + b3[...]
        o_ref[...] = jax.nn.sigmoid(h)


def classifier(flat, w1, b1, w2, b2, w3, b3, bk=4096):
    kp = ((flat.shape[1] + bk - 1) // bk) * bk
    flat = jnp.pad(flat, ((0, 0), (0, kp - flat.shape[1])))
    w1 = jnp.pad(w1, ((0, kp - w1.shape[0]), (0, 0)))
    m, kdim = flat.shape
    d1 = w1.shape[1]
    nk = kdim // bk
    full = lambda a: pl.BlockSpec(a.shape, lambda k: tuple(0 for _ in a.shape))
    b1r, b2r, b3r = b1.reshape(1, -1), b2.reshape(1, -1), b3.reshape(1, -1)
    return pl.pallas_call(
        functools.partial(_cf_kernel, nk=nk),
        grid=(nk,),
        in_specs=[
            pl.BlockSpec((m, bk), lambda k: (0, k)),
            pl.BlockSpec((bk, d1), lambda k: (k, 0)),
            full(b1r), full(w2), full(b2r), full(w3), full(b3r),
        ],
        out_specs=pl.BlockSpec((m, 2), lambda k: (0, 0)),
        out_shape=jax.ShapeDtypeStruct((m, 2), jnp.float32),
        scratch_shapes=[pltpu.VMEM((m, d1), jnp.float32)],
    )(flat, w1, b1r, w2, b2r, w3, b3r)


def _gru_kernel(gi0_ref, h0_ref, h1_ref, whh0, bhh0, wih1, bih1, whh1, bhh1,
                y_ref, h0o_ref, h1o_ref, *, T):
    def step(t, carry):
        h0, h1 = carry
        gi = gi0_ref[pl.ds(t, 1)][0]
        gh = jnp.dot(h0, whh0[...], preferred_element_type=jnp.float32) \
            + bhh0[...]
        i_r, i_z, i_n = jnp.split(gi, 3, axis=-1)
        h_r, h_z, h_n = jnp.split(gh, 3, axis=-1)
        r = jax.nn.sigmoid(i_r + h_r)
        z = jax.nn.sigmoid(i_z + h_z)
        nn = jnp.tanh(i_n + r * h_n)
        h0n = (1.0 - z) * nn + z * h0

        gi1 = jnp.dot(h0n, wih1[...], preferred_element_type=jnp.float32) \
            + bih1[...]
        gh1 = jnp.dot(h1, whh1[...], preferred_element_type=jnp.float32) \
            + bhh1[...]
        i_r1, i_z1, i_n1 = jnp.split(gi1, 3, axis=-1)
        h_r1, h_z1, h_n1 = jnp.split(gh1, 3, axis=-1)
        r1 = jax.nn.sigmoid(i_r1 + h_r1)
        z1 = jax.nn.sigmoid(i_z1 + h_z1)
        nn1 = jnp.tanh(i_n1 + r1 * h_n1)
        h1n = (1.0 - z1) * nn1 + z1 * h1

        y_ref[pl.ds(t, 1)] = h1n[None]
        return (h0n, h1n)

    h0, h1 = jax.lax.fori_loop(0, T, step, (h0_ref[...], h1_ref[...]))
    h0o_ref[...] = h0
    h1o_ref[...] = h1


def gru2(gi0, h00, h01, whh0, bhh0, wih1, bih1, whh1, bhh1):
    T = gi0.shape[0]
    full = lambda a: pl.BlockSpec(a.shape, lambda: tuple(0 for _ in a.shape))
    args = [gi0, h00, h01, whh0.T, bhh0.reshape(1, -1), wih1.T,
            bih1.reshape(1, -1), whh1.T, bhh1.reshape(1, -1)]
    return pl.pallas_call(
        functools.partial(_gru_kernel, T=T),
        in_specs=[full(a) for a in args],
        out_specs=[
            pl.BlockSpec((T, B, GH), lambda: (0, 0, 0)),
            pl.BlockSpec((B, GH), lambda: (0, 0)),
            pl.BlockSpec((B, GH), lambda: (0, 0)),
        ],
        out_shape=[
            jax.ShapeDtypeStruct((T, B, GH), jnp.float32),
            jax.ShapeDtypeStruct((B, GH), jnp.float32),
            jax.ShapeDtypeStruct((B, GH), jnp.float32),
        ],
    )(*args)


CE = 128
ACC = 10240
PT = ACC // 16


def _sc_edge_agg(tables, als_pk, ald_pk, src3, dst3, lanes, ereal):
    NP = len(tables)
    C = tables[0].shape[1]
    NCH = src3.shape[1]
    NSL = C // 16

    mesh = plsc.VectorSubcoreMesh(core_axis_name="c", subcore_axis_name="s")
    znum = jnp.zeros((ACC, C), jnp.float32)
    zden = jnp.zeros((ACC, 16), jnp.float32)

    @functools.partial(
        pl.kernel, mesh=mesh,
        compiler_params=pltpu.CompilerParams(use_tc_tiling_on_sc=False),
        out_type=[jax.ShapeDtypeStruct((2 * NP * ACC, C), jnp.float32),
                  jax.ShapeDtypeStruct((2 * ACC, 16), jnp.float32)],
        scratch_types=[
            pltpu.VMEM((NCH, CE), jnp.int32),
            pltpu.VMEM((NCH, CE), jnp.int32),
            pltpu.VMEM((CE, 16), jnp.float32),
            pltpu.VMEM((CE, 16), jnp.float32),
            pltpu.VMEM((CE, 16), jnp.float32),
            pltpu.VMEM((CE, C), jnp.float32),
            pltpu.VMEM_SHARED((ACC, C), jnp.float32),
            pltpu.VMEM_SHARED((ACC, 16), jnp.float32),
            pltpu.SemaphoreType.DMA,
            pltpu.SemaphoreType.DMA,
            pltpu.SemaphoreType.DMA,
        ],
    )
    def body(src3_r, dst3_r, als_r, ald_r, *rest):
        tabs = rest[:NP]
        znum_r, zden_r = rest[NP], rest[NP + 1]
        outn_r, outd_r = rest[NP + 2], rest[NP + 3]
        (srcb, dstb, arows, brows, wv, rows, accn, accd,
         s0, s1, s2) = rest[NP + 4:]
        cid = jax.lax.axis_index("c")
        sid = jax.lax.axis_index("s")
        wid = cid * 16 + sid
        _LANE_IOTA = jax.lax.iota(jnp.int32, 16)
        pltpu.sync_copy(src3_r.at[wid], srcb)
        pltpu.sync_copy(dst3_r.at[wid], dstb)

        @pl.when(sid == 0)
        def _():
            pltpu.sync_copy(zden_r, accd)

        for p in range(NP):
            @pl.when(sid == 0)
            def _():
                pltpu.sync_copy(znum_r, accn)
            plsc.subcore_barrier()

            def chunk(j, carry):
                cs = pltpu.async_copy(als_r.at[srcb.at[j]], arows, s0)
                cd = pltpu.async_copy(ald_r.at[dstb.at[j]], brows, s1)
                ct = pltpu.async_copy(tabs[p].at[srcb.at[j]], rows, s2)
                cs.wait()
                cd.wait()
                base = (wid * NCH + j) * CE

                def wrow(r, c2):
                    e = arows[r] + brows[r]
                    e = jnp.maximum(e, 0.2 * e)
                    w = jnp.exp(e)
                    w = jnp.where(base + r < ereal, w, jnp.zeros_like(w))
                    wv[r] = w
                    return c2
                jax.lax.fori_loop(0, CE, wrow, 0)
                ct.wait()

                def srow(r, c2):
                    wrow = wv[r]
                    for s in range(NSL):
                        wscal = wrow[lanes[p][s]]
                        rows[r, pl.ds(s * 16, 16)] = (
                            rows[r, pl.ds(s * 16, 16)] * wscal)
                    return c2
                jax.lax.fori_loop(0, CE, srow, 0)
                pltpu.sync_copy(rows, accn.at[dstb.at[j]], add=True)
                if p == 0:
                    pltpu.sync_copy(wv, accd.at[dstb.at[j]], add=True)
                return carry
            jax.lax.fori_loop(0, NCH, chunk, 0)
            plsc.subcore_barrier()
            pltpu.sync_copy(
                accn.at[pl.ds(sid * PT, PT)],
                outn_r.at[pl.ds((cid * NP + p) * ACC + sid * PT, PT)])
            plsc.subcore_barrier()
        pltpu.sync_copy(accd.at[pl.ds(sid * PT, PT)],
                        outd_r.at[pl.ds(cid * ACC + sid * PT, PT)])

    return body(src3, dst3, als_pk, ald_pk, *tables, znum, zden)


def _edge_aggregate(h, als, ald, src3, dst3, heads, outc, ereal):
    F = heads * outc
    als_pk = jnp.zeros((NT, 16), jnp.float32).at[:, :heads].set(als)
    ald_pk = jnp.zeros((NT, 16), jnp.float32).at[:, :heads].set(ald)
    if F <= 16:
        C = 16
        tables = [jnp.pad(h, ((0, 0), (0, C - F)))] if C != F else [h]
        lanes = [[0] * (C // 16)]
    else:
        C = 64
        NPp = F // C
        tables = [h[:, p * C:(p + 1) * C] for p in range(NPp)]
        lanes = [[(p * C + s * 16) // outc for s in range(C // 16)]
                 for p in range(NPp)]
    outn, outd = _sc_edge_agg(tables, als_pk, ald_pk, src3, dst3, lanes,
                              ereal)
    NP = len(tables)
    C = tables[0].shape[1]
    outn = outn.reshape(2, NP, ACC, C).sum(axis=0)[:, :NT]
    num = jnp.concatenate([outn[p] for p in range(NP)], axis=1)[:, :F]
    den = outd.reshape(2, ACC, 16).sum(axis=0)[:NT, :heads]
    out = num.reshape(NT, heads, outc) / (den + 1e-16)[:, :, None]
    return out.reshape(NT, F)


def _gat(xh, s3, d3, W, a_s, a_d, bias, heads, outc, ereal):
    F = heads * outc
    As = jnp.zeros((F, HD + 2), jnp.float32)
    Ad = jnp.zeros((F, HD + 2), jnp.float32)
    As = As.at[jnp.arange(F), jnp.repeat(jnp.arange(heads), outc)].set(
        a_s.reshape(F))
    Ad = Ad.at[jnp.arange(F), jnp.repeat(jnp.arange(heads), outc)].set(
        a_d.reshape(F))
    Wcat = jnp.concatenate([W, W @ As, W @ Ad], axis=1)
    hcat = matmul(xh, Wcat, bm=2000)
    h = hcat[:, :F]
    als = hcat[:, F:F + heads]
    ald = hcat[:, F + HD + 2:F + HD + 2 + heads]
    out = _edge_aggregate(h, als, ald, s3, d3, heads, outc, ereal)
    return out + bias


def kernel(x, adj, H_, params):
    p = params
    E = adj.shape[2]
    ereal = E + NT
    nch = -(-ereal // (32 * CE))
    ep_pad = 32 * nch * CE
    loop = jnp.arange(NT, dtype=jnp.int32)
    sd = []
    for m in range(MODES):
        s_full = jnp.concatenate([adj[m, 0], loop])
        d_full = jnp.concatenate([adj[m, 1], loop])
        s3 = jnp.pad(s_full, (0, ep_pad - ereal)).reshape(32, nch, CE)
        d3 = jnp.pad(d_full, (0, ep_pad - ereal)).reshape(32, nch, CE)
        sd.append((s3, d3))

    cat_t = None
    for m in range(MODES):
        mt = jax.lax.dynamic_slice_in_dim(x, m * NT, NT, 0)
        s, d = sd[m]
        g = _gat(mt, s, d, p['g1_W'][m], p['g1_as'][m], p['g1_ad'][m],
                 p['g1_b'][m], HD, 32, ereal)
        g = _gat(g, s, d, p['g2_W'][m], p['g2_as'][m], p['g2_ad'][m],
                 p['g2_b'][m], 1, EM, ereal)
        lf = mlp3(mt, p['nf_W1'], p['nf_b1'], p['nf_W2'], p['nf_b2'],
                  p['nf_W3'], p['nf_b3'])
        cat = matmul(jnp.concatenate([g, lf], axis=1), p['catP'][m])
        cat_t = cat if cat_t is None else cat_t + cat

    gi0 = matmul(cat_t, p['gru_Wih0'].T, p['gru_bih0'])
    gi0 = gi0.reshape(B, NODES, 3 * GH).transpose(1, 0, 2)
    y, h0T, h1T = gru2(gi0, H_[0], H_[1], p['gru_Whh0'], p['gru_bhh0'],
                       p['gru_Wih1'], p['gru_bih1'], p['gru_Whh1'],
                       p['gru_bhh1'])
    new_H = jnp.stack([h0T, h1T], axis=0)

    flat = y.transpose(1, 0, 2).reshape(B, NODES * GH)
    cf_out = classifier(flat, p['cf_W1'], p['cf_b1'], p['cf_W2'], p['cf_b2'],
                        p['cf_W3'], p['cf_b3'])

    rl = mlp3(cat_t, p['ml_W1'], p['ml_b1'], p['ml_W2'], p['ml_b2'],
              p['ml_W3'], p['ml_b3'])

    recs = []
    for m in range(MODES):
        s, d = sd[m]
        r = _gat(rl, s, d, p['r1_W'][m], p['r1_as'][m], p['r1_ad'][m],
                 p['r1_b'][m], HD, 32, ereal)
        r = _gat(r, s, d, p['r2_W'][m], p['r2_as'][m], p['r2_ad'][m],
                 p['r2_b'][m], 1, SL, ereal)
        recs.append(r)
    rec_out = jnp.concatenate(recs, axis=0)
    return (cf_out, rec_out, new_H)


# speedup vs baseline: 35.7688x; 1.0721x over previous
import functools
import jax
import jax.numpy as jnp
from jax.experimental import pallas as pl
from jax.experimental.pallas import tpu as pltpu
from jax.experimental.pallas import tpu_sc as plsc

MODES = 2
NODES = 2500
B = 4
SL = 128
NT = NODES * B
HD = 6
GH = 16
EM = 4


def _mm_kernel(x_ref, w_ref, b_ref, o_ref, *, act, nk):
    k = pl.program_id(1)

    @pl.when(k == 0)
    def _():
        o_ref[...] = jnp.zeros_like(o_ref)

    o_ref[...] += jnp.dot(x_ref[...], w_ref[...],
                          preferred_element_type=jnp.float32)

    @pl.when(k == nk - 1)
    def _():
        acc = o_ref[...] + b_ref[...]
        if act == 'relu':
            acc = jnp.maximum(acc, 0.0)
        elif act == 'sigmoid':
            acc = jax.nn.sigmoid(acc)
        o_ref[...] = acc


def matmul(x, w, b=None, act='none', bm=None, bk=None):
    m, kdim = x.shape
    n = w.shape[1]
    if b is None:
        b = jnp.zeros((n,), jnp.float32)
    b = b.reshape(1, n)
    bm = bm or m
    bk = bk or kdim
    grid = (m // bm, kdim // bk)
    return pl.pallas_call(
        functools.partial(_mm_kernel, act=act, nk=grid[1]),
        grid=grid,
        in_specs=[
            pl.BlockSpec((bm, bk), lambda i, k: (i, k)),
            pl.BlockSpec((bk, n), lambda i, k: (k, 0)),
            pl.BlockSpec((1, n), lambda i, k: (0, 0)),
        ],
        out_specs=pl.BlockSpec((bm, n), lambda i, k: (i, 0)),
        out_shape=jax.ShapeDtypeStruct((m, n), jnp.float32),
    )(x, w, b)


def _mlp3_kernel(x_ref, w1, b1, w2, b2, w3, b3, o_ref, *, final_act):
    h = jnp.maximum(jnp.dot(x_ref[...], w1[...],
                            preferred_element_type=jnp.float32) + b1[...], 0.0)
    h = jnp.maximum(jnp.dot(h, w2[...],
                            preferred_element_type=jnp.float32) + b2[...], 0.0)
    h = jnp.dot(h, w3[...], preferred_element_type=jnp.float32) + b3[...]
    if final_act == 'sigmoid':
        h = jax.nn.sigmoid(h)
    o_ref[...] = h


def mlp3(x, w1, b1, w2, b2, w3, b3, final_act='none', bm=2000):
    m, kdim = x.shape
    d1 = w1.shape[1]
    d2 = w2.shape[1]
    n = w3.shape[1]
    grid = (m // bm,)
    full = lambda a: pl.BlockSpec(a.shape, lambda i: tuple(0 for _ in a.shape))
    b1r, b2r, b3r = b1.reshape(1, d1), b2.reshape(1, d2), b3.reshape(1, n)
    return pl.pallas_call(
        functools.partial(_mlp3_kernel, final_act=final_act),
        grid=grid,
        in_specs=[
            pl.BlockSpec((bm, kdim), lambda i: (i, 0)),
            full(w1), full(b1r), full(w2), full(b2r), full(w3), full(b3r),
        ],
        out_specs=pl.BlockSpec((bm, n), lambda i: (i, 0)),
        out_shape=jax.ShapeDtypeStruct((m, n), jnp.float32),
    )(x, w1, b1r, w2, b2r, w3, b3r)


def _cf_kernel(x_ref, w1_ref, b1, w2, b2, w3, b3, o_ref, acc, *, nk):
    k = pl.program_id(0)

    @pl.when(k == 0)
    def _():
        acc[...] = jnp.zeros_like(acc)

    acc[...] += jnp.dot(x_ref[...], w1_ref[...],
                        preferred_element_type=jnp.float32)

    @pl.when(k == nk - 1)
    def _():
        h = jnp.maximum(acc[...] + b1[...], 0.0)
        h = jnp.maximum(jnp.dot(h, w2[...],
                                preferred_element_type=jnp.float32) + b2[...],
                        0.0)
        h = jnp.dot(h, w3[...], preferred_element_type=jnp.float32) + b3[...]
        o_ref[...] = jax.nn.sigmoid(h)


def classifier(flat, w1, b1, w2, b2, w3, b3, bk=4096):
    kp = ((flat.shape[1] + bk - 1) // bk) * bk
    flat = jnp.pad(flat, ((0, 0), (0, kp - flat.shape[1])))
    w1 = jnp.pad(w1, ((0, kp - w1.shape[0]), (0, 0)))
    m, kdim = flat.shape
    d1 = w1.shape[1]
    nk = kdim // bk
    full = lambda a: pl.BlockSpec(a.shape, lambda k: tuple(0 for _ in a.shape))
    b1r, b2r, b3r = b1.reshape(1, -1), b2.reshape(1, -1), b3.reshape(1, -1)
    return pl.pallas_call(
        functools.partial(_cf_kernel, nk=nk),
        grid=(nk,),
        in_specs=[
            pl.BlockSpec((m, bk), lambda k: (0, k)),
            pl.BlockSpec((bk, d1), lambda k: (k, 0)),
            full(b1r), full(w2), full(b2r), full(w3), full(b3r),
        ],
        out_specs=pl.BlockSpec((m, 2), lambda k: (0, 0)),
        out_shape=jax.ShapeDtypeStruct((m, 2), jnp.float32),
        scratch_shapes=[pltpu.VMEM((m, d1), jnp.float32)],
    )(flat, w1, b1r, w2, b2r, w3, b3r)


def _gru_kernel(gi0_ref, h0_ref, h1_ref, whh0, bhh0, wih1, bih1, whh1, bhh1,
                y_ref, h0o_ref, h1o_ref, *, T):
    def step(t, carry):
        h0, h1 = carry
        gi = gi0_ref[pl.ds(t, 1)][0]
        gh = jnp.dot(h0, whh0[...], preferred_element_type=jnp.float32) \
            + bhh0[...]
        i_r, i_z, i_n = jnp.split(gi, 3, axis=-1)
        h_r, h_z, h_n = jnp.split(gh, 3, axis=-1)
        r = jax.nn.sigmoid(i_r + h_r)
        z = jax.nn.sigmoid(i_z + h_z)
        nn = jnp.tanh(i_n + r * h_n)
        h0n = (1.0 - z) * nn + z * h0

        gi1 = jnp.dot(h0n, wih1[...], preferred_element_type=jnp.float32) \
            + bih1[...]
        gh1 = jnp.dot(h1, whh1[...], preferred_element_type=jnp.float32) \
            + bhh1[...]
        i_r1, i_z1, i_n1 = jnp.split(gi1, 3, axis=-1)
        h_r1, h_z1, h_n1 = jnp.split(gh1, 3, axis=-1)
        r1 = jax.nn.sigmoid(i_r1 + h_r1)
        z1 = jax.nn.sigmoid(i_z1 + h_z1)
        nn1 = jnp.tanh(i_n1 + r1 * h_n1)
        h1n = (1.0 - z1) * nn1 + z1 * h1

        y_ref[pl.ds(t, 1)] = h1n[None]
        return (h0n, h1n)

    h0, h1 = jax.lax.fori_loop(0, T, step, (h0_ref[...], h1_ref[...]))
    h0o_ref[...] = h0
    h1o_ref[...] = h1


def gru2(gi0, h00, h01, whh0, bhh0, wih1, bih1, whh1, bhh1):
    T = gi0.shape[0]
    full = lambda a: pl.BlockSpec(a.shape, lambda: tuple(0 for _ in a.shape))
    args = [gi0, h00, h01, whh0.T, bhh0.reshape(1, -1), wih1.T,
            bih1.reshape(1, -1), whh1.T, bhh1.reshape(1, -1)]
    return pl.pallas_call(
        functools.partial(_gru_kernel, T=T),
        in_specs=[full(a) for a in args],
        out_specs=[
            pl.BlockSpec((T, B, GH), lambda: (0, 0, 0)),
            pl.BlockSpec((B, GH), lambda: (0, 0)),
            pl.BlockSpec((B, GH), lambda: (0, 0)),
        ],
        out_shape=[
            jax.ShapeDtypeStruct((T, B, GH), jnp.float32),
            jax.ShapeDtypeStruct((B, GH), jnp.float32),
            jax.ShapeDtypeStruct((B, GH), jnp.float32),
        ],
    )(*args)


CE = 128
ACC = 10240
PT = ACC // 16


def _sc_edge_agg(tables, als_pk, ald_pk, src3, dst3, lanes, ereal):
    NP = len(tables)
    C = tables[0].shape[1]
    NCH = src3.shape[1]
    NSL = C // 16

    mesh = plsc.VectorSubcoreMesh(core_axis_name="c", subcore_axis_name="s")
    znum = jnp.zeros((ACC, C), jnp.float32)
    zden = jnp.zeros((ACC, 16), jnp.float32)

    @functools.partial(
        pl.kernel, mesh=mesh,
        compiler_params=pltpu.CompilerParams(use_tc_tiling_on_sc=False),
        out_type=[jax.ShapeDtypeStruct((2 * NP * ACC, C), jnp.float32),
                  jax.ShapeDtypeStruct((2 * ACC, 16), jnp.float32)],
        scratch_types=[
            pltpu.VMEM((NCH, CE), jnp.int32),
            pltpu.VMEM((NCH, CE), jnp.int32),
            pltpu.VMEM((2, CE, 16), jnp.float32),
            pltpu.VMEM((2, CE, 16), jnp.float32),
            pltpu.VMEM((CE, 16), jnp.float32),
            pltpu.VMEM((2, CE, C), jnp.float32),
            pltpu.VMEM_SHARED((ACC, C), jnp.float32),
            pltpu.VMEM_SHARED((ACC, 16), jnp.float32),
            pltpu.SemaphoreType.DMA,
            pltpu.SemaphoreType.DMA,
            pltpu.SemaphoreType.DMA,
            pltpu.SemaphoreType.DMA,
            pltpu.SemaphoreType.DMA,
            pltpu.SemaphoreType.DMA,
        ],
    )
    def body(src3_r, dst3_r, als_r, ald_r, *rest):
        tabs = rest[:NP]
        znum_r, zden_r = rest[NP], rest[NP + 1]
        outn_r, outd_r = rest[NP + 2], rest[NP + 3]
        (srcb, dstb, arows, brows, wv, rows, accn, accd,
         sa0, sb0, st0, sa1, sb1, st1) = rest[NP + 4:]
        sems = ((sa0, sb0, st0), (sa1, sb1, st1))
        cid = jax.lax.axis_index("c")
        sid = jax.lax.axis_index("s")
        wid = cid * 16 + sid
        pltpu.sync_copy(src3_r.at[wid], srcb)
        pltpu.sync_copy(dst3_r.at[wid], dstb)

        @pl.when(sid == 0)
        def _():
            pltpu.sync_copy(zden_r, accd)

        for p in range(NP):
            tab = tabs[p]

            def start(j, k):
                sa, sb, st = sems[k]
                pltpu.async_copy(als_r.at[srcb.at[j]], arows.at[k], sa)
                pltpu.async_copy(ald_r.at[dstb.at[j]], brows.at[k], sb)
                pltpu.async_copy(tab.at[srcb.at[j]], rows.at[k], st)

            def wait_ab(k):
                sa, sb, _ = sems[k]
                pltpu.make_async_copy(als_r.at[srcb.at[0]], arows.at[k],
                                      sa).wait()
                pltpu.make_async_copy(ald_r.at[dstb.at[0]], brows.at[k],
                                      sb).wait()

            def wait_t(k):
                _, _, st = sems[k]
                pltpu.make_async_copy(tab.at[srcb.at[0]], rows.at[k],
                                      st).wait()

            def process(j, k):
                wait_ab(k)
                base = (wid * NCH + j) * CE

                def wrow(r, c2):
                    e = arows[k, r] + brows[k, r]
                    e = jnp.maximum(e, 0.2 * e)
                    w = jnp.exp(e)
                    w = jnp.where(base + r < ereal, w, jnp.zeros_like(w))
                    wv[r] = w
                    return c2
                jax.lax.fori_loop(0, CE, wrow, 0)
                wait_t(k)

                def srow(r, c2):
                    wr = wv[r]
                    for s in range(NSL):
                        wscal = wr[lanes[p][s]]
                        rows[k, r, pl.ds(s * 16, 16)] = (
                            rows[k, r, pl.ds(s * 16, 16)] * wscal)
                    return c2
                jax.lax.fori_loop(0, CE, srow, 0)
                pltpu.sync_copy(rows.at[k], accn.at[dstb.at[j]], add=True)
                if p == 0:
                    pltpu.sync_copy(wv, accd.at[dstb.at[j]], add=True)

            @pl.when(sid == 0)
            def _():
                pltpu.sync_copy(znum_r, accn)
            plsc.subcore_barrier()

            start(0, 0)

            def chunk2(jj, carry):
                j0 = 2 * jj
                start(j0 + 1, 1)
                process(j0, 0)
                start(jnp.minimum(j0 + 2, NCH - 2), 0)
                process(j0 + 1, 1)
                return carry
            jax.lax.fori_loop(0, NCH // 2, chunk2, 0)
            wait_ab(0)
            wait_t(0)
            plsc.subcore_barrier()
            pltpu.sync_copy(
                accn.at[pl.ds(sid * PT, PT)],
                outn_r.at[pl.ds((cid * NP + p) * ACC + sid * PT, PT)])
            plsc.subcore_barrier()
        pltpu.sync_copy(accd.at[pl.ds(sid * PT, PT)],
                        outd_r.at[pl.ds(cid * ACC + sid * PT, PT)])

    return body(src3, dst3, als_pk, ald_pk, *tables, znum, zden)


def _edge_aggregate(h, als, ald, src3, dst3, heads, outc, ereal):
    F = heads * outc
    als_pk = jnp.zeros((NT, 16), jnp.float32).at[:, :heads].set(als)
    ald_pk = jnp.zeros((NT, 16), jnp.float32).at[:, :heads].set(ald)
    if F <= 16:
        C = 16
        tables = [jnp.pad(h, ((0, 0), (0, C - F)))] if C != F else [h]
        lanes = [[0] * (C // 16)]
    else:
        C = 64
        NPp = F // C
        tables = [h[:, p * C:(p + 1) * C] for p in range(NPp)]
        lanes = [[(p * C + s * 16) // outc for s in range(C // 16)]
                 for p in range(NPp)]
    outn, outd = _sc_edge_agg(tables, als_pk, ald_pk, src3, dst3, lanes,
                              ereal)
    NP = len(tables)
    C = tables[0].shape[1]
    outn = outn.reshape(2, NP, ACC, C).sum(axis=0)[:, :NT]
    num = jnp.concatenate([outn[p] for p in range(NP)], axis=1)[:, :F]
    den = outd.reshape(2, ACC, 16).sum(axis=0)[:NT, :heads]
    out = num.reshape(NT, heads, outc) / (den + 1e-16)[:, :, None]
    return out.reshape(NT, F)


def _gat(xh, s3, d3, W, a_s, a_d, bias, heads, outc, ereal):
    F = heads * outc
    As = jnp.zeros((F, HD + 2), jnp.float32)
    Ad = jnp.zeros((F, HD + 2), jnp.float32)
    As = As.at[jnp.arange(F), jnp.repeat(jnp.arange(heads), outc)].set(
        a_s.reshape(F))
    Ad = Ad.at[jnp.arange(F), jnp.repeat(jnp.arange(heads), outc)].set(
        a_d.reshape(F))
    Wcat = jnp.concatenate([W, W @ As, W @ Ad], axis=1)
    hcat = matmul(xh, Wcat, bm=2000)
    h = hcat[:, :F]
    als = hcat[:, F:F + heads]
    ald = hcat[:, F + HD + 2:F + HD + 2 + heads]
    out = _edge_aggregate(h, als, ald, s3, d3, heads, outc, ereal)
    return out + bias


def kernel(x, adj, H_, params):
    p = params
    E = adj.shape[2]
    ereal = E + NT
    nch = -(-ereal // (32 * CE))
    nch += nch % 2
    ep_pad = 32 * nch * CE
    loop = jnp.arange(NT, dtype=jnp.int32)
    sd = []
    for m in range(MODES):
        s_full = jnp.concatenate([adj[m, 0], loop])
        d_full = jnp.concatenate([adj[m, 1], loop])
        s3 = jnp.pad(s_full, (0, ep_pad - ereal)).reshape(32, nch, CE)
        d3 = jnp.pad(d_full, (0, ep_pad - ereal)).reshape(32, nch, CE)
        sd.append((s3, d3))

    cat_t = None
    for m in range(MODES):
        mt = jax.lax.dynamic_slice_in_dim(x, m * NT, NT, 0)
        s, d = sd[m]
        g = _gat(mt, s, d, p['g1_W'][m], p['g1_as'][m], p['g1_ad'][m],
                 p['g1_b'][m], HD, 32, ereal)
        g = _gat(g, s, d, p['g2_W'][m], p['g2_as'][m], p['g2_ad'][m],
                 p['g2_b'][m], 1, EM, ereal)
        lf = mlp3(mt, p['nf_W1'], p['nf_b1'], p['nf_W2'], p['nf_b2'],
                  p['nf_W3'], p['nf_b3'])
        cat = matmul(jnp.concatenate([g, lf], axis=1), p['catP'][m])
        cat_t = cat if cat_t is None else cat_t + cat

    gi0 = matmul(cat_t, p['gru_Wih0'].T, p['gru_bih0'])
    gi0 = gi0.reshape(B, NODES, 3 * GH).transpose(1, 0, 2)
    y, h0T, h1T = gru2(gi0, H_[0], H_[1], p['gru_Whh0'], p['gru_bhh0'],
                       p['gru_Wih1'], p['gru_bih1'], p['gru_Whh1'],
                       p['gru_bhh1'])
    new_H = jnp.stack([h0T, h1T], axis=0)

    flat = y.transpose(1, 0, 2).reshape(B, NODES * GH)
    cf_out = classifier(flat, p['cf_W1'], p['cf_b1'], p['cf_W2'], p['cf_b2'],
                        p['cf_W3'], p['cf_b3'])

    rl = mlp3(cat_t, p['ml_W1'], p['ml_b1'], p['ml_W2'], p['ml_b2'],
              p['ml_W3'], p['ml_b3'])

    recs = []
    for m in range(MODES):
        s, d = sd[m]
        r = _gat(rl, s, d, p['r1_W'][m], p['r1_as'][m], p['r1_ad'][m],
                 p['r1_b'][m], HD, 32, ereal)
        r = _gat(r, s, d, p['r2_W'][m], p['r2_as'][m], p['r2_ad'][m],
                 p['r2_b'][m], 1, SL, ereal)
        recs.append(r)
    rec_out = jnp.concatenate(recs, axis=0)
    return (cf_out, rec_out, new_H)
